# Initial kernel scaffold; baseline (speedup 1.0000x reference)
import jax
import jax.numpy as jnp
from jax.experimental import pallas as pl


def kernel(x, edge_index, edge_attr, W_t, b_t, W1, b1, W2, b2, Wc, bc):
    raise NotImplementedError("write your pallas kernel here")


# speedup vs baseline: 7.7721x; 7.7721x over previous
import functools

import jax
import jax.numpy as jnp
from jax import lax
from jax.experimental import pallas as pl
from jax.experimental.pallas import tpu as pltpu
from jax.experimental.pallas import tpu_sc as plsc

N = 10000
E = 320000
NPAD = 10240
ROWS_PT = 640
W_WIN = 80

_mesh = plsc.VectorSubcoreMesh(core_axis_name="c", subcore_axis_name="s")


def _deg_kernel(ea_hbm, dst_hbm, ew_hbm, degp_hbm, eab, ewb, dstb, zb, dacc):
    c = lax.axis_index("c")
    s = lax.axis_index("s")

    for i in range(5):
        zb[pl.ds(16 * i, 16)] = jnp.zeros((16,), jnp.float32)
    for i in range(ROWS_PT // W_WIN):
        pltpu.sync_copy(zb, dacc.at[pl.ds(s * ROWS_PT + i * W_WIN, W_WIN)])
    plsc.subcore_barrier()

    ept = E // 32
    base = (c * 16 + s) * ept

    def body(w, _):
        off = base + w * W_WIN
        pltpu.sync_copy(ea_hbm.at[pl.ds(off, W_WIN)], eab)
        pltpu.sync_copy(dst_hbm.at[pl.ds(off, W_WIN)], dstb)
        for i in range(W_WIN // 16):
            ewb[pl.ds(16 * i, 16)] = jnp.abs(eab[pl.ds(16 * i, 16)])
        pltpu.sync_copy(ewb, ew_hbm.at[pl.ds(off, W_WIN)])
        pltpu.sync_copy(ewb, dacc.at[dstb], add=True)
        return 0

    lax.fori_loop(0, ept // W_WIN, body, 0)
    plsc.subcore_barrier()
    pltpu.sync_copy(dacc.at[pl.ds(s * ROWS_PT, ROWS_PT)],
                    degp_hbm.at[c, pl.ds(s * ROWS_PT, ROWS_PT)])


def _run_deg(ea, dst):
    f = functools.partial(
        pl.kernel,
        mesh=_mesh,
        out_type=[
            jax.ShapeDtypeStruct((E,), jnp.float32),
            jax.ShapeDtypeStruct((2, NPAD), jnp.float32),
        ],
        scratch_types=[
            pltpu.VMEM((W_WIN,), jnp.float32),
            pltpu.VMEM((W_WIN,), jnp.float32),
            pltpu.VMEM((W_WIN,), jnp.int32),
            pltpu.VMEM((W_WIN,), jnp.float32),
            pltpu.VMEM_SHARED((NPAD,), jnp.float32),
        ],
    )(_deg_kernel)
    return f(ea, dst)


def _agg_body(edge_split, p_hbm, src_hbm, dst_hbm, ew_hbm, out_hbm, srcb,
              dstb, ewb, rows, acc, sem):
    Dh = 128
    c = lax.axis_index("c")
    s = lax.axis_index("s")

    for k in range(W_WIN):
        for j in range(Dh // 16):
            rows[k, pl.ds(16 * j, 16)] = jnp.zeros((16,), jnp.float32)
    for i in range(ROWS_PT // W_WIN):
        pltpu.sync_copy(rows, acc.at[pl.ds(s * ROWS_PT + i * W_WIN, W_WIN)])
    plsc.subcore_barrier()

    if edge_split:
        ept = E // 32
        base = (c * 16 + s) * ept
        coff = 0
    else:
        ept = E // 16
        base = s * ept
        coff = c * N

    def body(w, _):
        off = base + w * W_WIN
        pltpu.sync_copy(src_hbm.at[pl.ds(off, W_WIN)], srcb)
        pltpu.sync_copy(dst_hbm.at[pl.ds(off, W_WIN)], dstb)
        pltpu.sync_copy(ew_hbm.at[pl.ds(off, W_WIN)], ewb)
        if not edge_split:
            for i in range(W_WIN // 16):
                srcb[pl.ds(16 * i, 16)] = srcb[pl.ds(16 * i, 16)] + coff
        pltpu.async_copy(p_hbm.at[srcb], rows, sem).wait()

        def scale(ci, _):
            ew16 = ewb[pl.ds(16 * ci, 16)]
            for l in range(16):
                ewv = jnp.full((16,), ew16[l], jnp.float32)
                k = 16 * ci + l
                for j in range(Dh // 16):
                    sl = pl.ds(16 * j, 16)
                    rows[k, sl] = rows[k, sl] * ewv
            return 0

        lax.fori_loop(0, W_WIN // 16, scale, 0)
        pltpu.sync_copy(rows, acc.at[dstb], add=True)
        return 0

    lax.fori_loop(0, ept // W_WIN, body, 0)
    plsc.subcore_barrier()

    @pl.when(s < 15)
    def _():
        pltpu.sync_copy(acc.at[pl.ds(s * ROWS_PT, ROWS_PT)],
                        out_hbm.at[c, pl.ds(s * ROWS_PT, ROWS_PT)])

    @pl.when(s == 15)
    def _():
        pltpu.sync_copy(acc.at[pl.ds(15 * ROWS_PT, N - 15 * ROWS_PT)],
                        out_hbm.at[c, pl.ds(15 * ROWS_PT, N - 15 * ROWS_PT)])


def _run_agg(p_flat, src, dst, ew, edge_split):
    f = functools.partial(
        pl.kernel,
        mesh=_mesh,
        out_type=jax.ShapeDtypeStruct((2, N, 128), jnp.float32),
        scratch_types=[
            pltpu.VMEM((W_WIN,), jnp.int32),
            pltpu.VMEM((W_WIN,), jnp.int32),
            pltpu.VMEM((W_WIN,), jnp.float32),
            pltpu.VMEM((W_WIN, 128), jnp.float32),
            pltpu.VMEM_SHARED((NPAD, 128), jnp.float32),
            pltpu.SemaphoreType.DMA,
        ],
    )(functools.partial(_agg_body, edge_split))
    return f(p_flat, src, dst, ew)


_RB = 1000


def _dinv_body(degp_ref, out_ref):
    out_ref[...] = lax.rsqrt(degp_ref[0] + degp_ref[1] + 1.0)


def _run_dinv(degp):
    out = pl.pallas_call(
        _dinv_body,
        in_specs=[pl.BlockSpec((2, NPAD // 128, 128), lambda: (0, 0, 0))],
        out_specs=pl.BlockSpec((NPAD // 128, 128), lambda: (0, 0)),
        out_shape=jax.ShapeDtypeStruct((NPAD // 128, 128), jnp.float32),
    )(degp.reshape(2, NPAD // 128, 128))
    return out.reshape(NPAD, 1)


def _stage1_body(x_ref, wt_ref, bt_ref, w1_ref, dinv_ref, p1_ref):
    h0 = jnp.maximum(
        jnp.dot(x_ref[...], wt_ref[...], preferred_element_type=jnp.float32)
        + bt_ref[...], 0.0)
    g = jnp.dot(h0, w1_ref[...], preferred_element_type=jnp.float32)
    p = dinv_ref[...] * g
    p1_ref[0] = p[:, :128]
    p1_ref[1] = p[:, 128:]


def _stage2_body(a1_ref, p1_ref, b1_ref, w2t_ref, w2b_ref, dinv_ref, p2_ref):
    dinv = dinv_ref[...]
    h_lo = jnp.maximum(dinv * (a1_ref[0] + p1_ref[0]) + b1_ref[0, :][None, :], 0.0)
    h_hi = jnp.maximum(dinv * (a1_ref[1] + p1_ref[1]) + b1_ref[1, :][None, :], 0.0)
    g = (jnp.dot(h_lo, w2t_ref[...], preferred_element_type=jnp.float32)
         + jnp.dot(h_hi, w2b_ref[...], preferred_element_type=jnp.float32))
    p2_ref[...] = dinv * g


def _stage3_body(a2_ref, p2_ref, b2_ref, wc_ref, bc_ref, dinv_ref, out_ref):
    dinv = dinv_ref[...]
    agg0 = a2_ref[0] + a2_ref[1]
    h2 = jnp.maximum(dinv * (agg0 + p2_ref[...]) + b2_ref[...], 0.0)
    out_ref[...] = (
        jnp.dot(h2, wc_ref[...], preferred_element_type=jnp.float32)
        + bc_ref[...])


def _full(shape):
    return pl.BlockSpec(shape, lambda i: tuple(0 for _ in shape))


def _run_stage1(x, W_t, b_t, W1, dinv):
    grid = (N // _RB,)
    return pl.pallas_call(
        _stage1_body,
        grid=grid,
        in_specs=[
            pl.BlockSpec((_RB, 128), lambda i: (i, 0)),
            _full((128, 128)),
            _full((1, 128)),
            _full((128, 256)),
            pl.BlockSpec((_RB, 1), lambda i: (i, 0)),
        ],
        out_specs=pl.BlockSpec((2, _RB, 128), lambda i: (0, i, 0)),
        out_shape=jax.ShapeDtypeStruct((2, N, 128), jnp.float32),
    )(x, W_t, b_t.reshape(1, 128), W1, dinv)


def _run_stage2(a1, p1, b1, W2, dinv):
    grid = (N // _RB,)
    return pl.pallas_call(
        _stage2_body,
        grid=grid,
        in_specs=[
            pl.BlockSpec((2, _RB, 128), lambda i: (0, i, 0)),
            pl.BlockSpec((2, _RB, 128), lambda i: (0, i, 0)),
            _full((2, 128)),
            _full((128, 128)),
            _full((128, 128)),
            pl.BlockSpec((_RB, 1), lambda i: (i, 0)),
        ],
        out_specs=pl.BlockSpec((_RB, 128), lambda i: (i, 0)),
        out_shape=jax.ShapeDtypeStruct((N, 128), jnp.float32),
    )(a1, p1, b1.reshape(2, 128), W2[:128], W2[128:], dinv)


def _run_stage3(a2, p2, b2, Wc, bc, dinv):
    grid = (N // _RB,)
    return pl.pallas_call(
        _stage3_body,
        grid=grid,
        in_specs=[
            pl.BlockSpec((2, _RB, 128), lambda i: (0, i, 0)),
            pl.BlockSpec((_RB, 128), lambda i: (i, 0)),
            _full((1, 128)),
            _full((128, 8)),
            _full((1, 8)),
            pl.BlockSpec((_RB, 1), lambda i: (i, 0)),
        ],
        out_specs=pl.BlockSpec((_RB, 8), lambda i: (i, 0)),
        out_shape=jax.ShapeDtypeStruct((N, 8), jnp.float32),
    )(a2, p2, b2.reshape(1, 128), Wc, bc.reshape(1, 8), dinv)


@jax.jit
def kernel(x, edge_index, edge_attr, W_t, b_t, W1, b1, W2, b2, Wc, bc):
    ea = edge_attr.reshape(E)
    src = edge_index[0]
    dst = edge_index[1]
    ew, degp = _run_deg(ea, dst)
    dinv = _run_dinv(degp)
    p1 = _run_stage1(x, W_t, b_t, W1, dinv)
    a1 = _run_agg(p1.reshape(2 * N, 128), src, dst, ew, edge_split=False)
    p2 = _run_stage2(a1, p1, b1, W2, dinv)
    a2 = _run_agg(p2, src, dst, ew, edge_split=True)
    return _run_stage3(a2, p2, b2, Wc, bc, dinv)


# speedup vs baseline: 18.8372x; 2.4237x over previous
import functools

import jax
import jax.numpy as jnp
from jax import lax
from jax.experimental import pallas as pl
from jax.experimental.pallas import tpu as pltpu
from jax.experimental.pallas import tpu_sc as plsc

N = 10000
E = 320000
NPAD = 10240
ROWS_PT = 640
W_WIN = 80
AGG_CH_FS = 4000
AGG_CH_ES = 2000

_mesh = plsc.VectorSubcoreMesh(core_axis_name="c", subcore_axis_name="s")


def _deg_kernel(ea_hbm, dst_hbm, ew_hbm, degp_hbm, eav, dstv, dstb, zb, dacc):
    c = lax.axis_index("c")
    s = lax.axis_index("s")
    ept = E // 32
    base = (c * 16 + s) * ept

    for i in range(5):
        zb[pl.ds(16 * i, 16)] = jnp.zeros((16,), jnp.float32)
    for i in range(ROWS_PT // W_WIN):
        pltpu.sync_copy(zb, dacc.at[pl.ds(s * ROWS_PT + i * W_WIN, W_WIN)])

    pltpu.sync_copy(ea_hbm.at[pl.ds(base, ept)], eav)
    pltpu.sync_copy(dst_hbm.at[pl.ds(base, ept)], dstv)

    def absbody(i, _):
        sl = pl.ds(16 * i, 16)
        eav[sl] = jnp.abs(eav[sl])
        return 0

    lax.fori_loop(0, ept // 16, absbody, 0)
    pltpu.sync_copy(eav, ew_hbm.at[pl.ds(base, ept)])
    plsc.subcore_barrier()

    def body(w, _):
        for i in range(W_WIN // 16):
            dstb[pl.ds(16 * i, 16)] = dstv[pl.ds(w * W_WIN + 16 * i, 16)]
        pltpu.sync_copy(eav.at[pl.ds(w * W_WIN, W_WIN)], dacc.at[dstb],
                        add=True)
        return 0

    lax.fori_loop(0, ept // W_WIN, body, 0)
    plsc.subcore_barrier()
    pltpu.sync_copy(dacc.at[pl.ds(s * ROWS_PT, ROWS_PT)],
                    degp_hbm.at[c, pl.ds(s * ROWS_PT, ROWS_PT)])


def _run_deg(ea, dst):
    f = functools.partial(
        pl.kernel,
        mesh=_mesh,
        out_type=[
            jax.ShapeDtypeStruct((E,), jnp.float32),
            jax.ShapeDtypeStruct((2, NPAD), jnp.float32),
        ],
        scratch_types=[
            pltpu.VMEM((E // 32,), jnp.float32),
            pltpu.VMEM((E // 32,), jnp.int32),
            pltpu.VMEM((W_WIN,), jnp.int32),
            pltpu.VMEM((W_WIN,), jnp.float32),
            pltpu.VMEM_SHARED((NPAD,), jnp.float32),
        ],
    )(_deg_kernel)
    return f(ea, dst)


def _agg_body(edge_split, p_hbm, src_hbm, dst_hbm, ew_hbm, out_hbm, srcv,
              dstv, ewv, dstb, rows0, rows1, acc, sem0, sem1):
    Dh = 128
    CH = AGG_CH_ES if edge_split else AGG_CH_FS
    c = lax.axis_index("c")
    s = lax.axis_index("s")

    if edge_split:
        ept = E // 32
        base = (c * 16 + s) * ept
    else:
        ept = E // 16
        base = s * ept
    nw = CH // W_WIN
    nch = ept // CH

    for k in range(W_WIN):
        for j in range(Dh // 16):
            rows0[k, pl.ds(16 * j, 16)] = jnp.zeros((16,), jnp.float32)
    for i in range(ROWS_PT // W_WIN):
        pltpu.sync_copy(rows0, acc.at[pl.ds(s * ROWS_PT + i * W_WIN, W_WIN)])
    plsc.subcore_barrier()

    def step(w, cur_rows, cur_sem, nxt_rows, nxt_sem):
        pltpu.make_async_copy(p_hbm.at[pl.ds(0, W_WIN)], cur_rows,
                              cur_sem).wait()

        @pl.when(w + 1 < nw)
        def _():
            pltpu.async_copy(
                p_hbm.at[srcv.at[pl.ds((w + 1) * W_WIN, W_WIN)]],
                nxt_rows, nxt_sem)

        def scale(ci, _):
            ew16 = ewv[pl.ds(w * W_WIN + 16 * ci, 16)]
            for l in range(16):
                ewb = jnp.full((16,), ew16[l], jnp.float32)
                k = 16 * ci + l
                for j in range(Dh // 16):
                    sl = pl.ds(16 * j, 16)
                    cur_rows[k, sl] = cur_rows[k, sl] * ewb
            return 0

        lax.fori_loop(0, W_WIN // 16, scale, 0)
        for i in range(W_WIN // 16):
            dstb[pl.ds(16 * i, 16)] = dstv[pl.ds(w * W_WIN + 16 * i, 16)]
        pltpu.sync_copy(cur_rows, acc.at[dstb], add=True)

    def chunk(ci, _):
        cbase = base + ci * CH
        pltpu.sync_copy(src_hbm.at[pl.ds(cbase, CH)], srcv)
        pltpu.sync_copy(dst_hbm.at[pl.ds(cbase, CH)], dstv)
        pltpu.sync_copy(ew_hbm.at[pl.ds(cbase, CH)], ewv)
        if not edge_split:
            coff = c * N

            def addoff(i, _):
                sl = pl.ds(16 * i, 16)
                srcv[sl] = srcv[sl] + coff
                return 0

            lax.fori_loop(0, CH // 16, addoff, 0)
        pltpu.async_copy(p_hbm.at[srcv.at[pl.ds(0, W_WIN)]], rows0, sem0)

        def body(w, _):
            @pl.when(w % 2 == 0)
            def _():
                step(w, rows0, sem0, rows1, sem1)

            @pl.when(w % 2 == 1)
            def _():
                step(w, rows1, sem1, rows0, sem0)

            return 0

        lax.fori_loop(0, nw, body, 0)
        return 0

    lax.fori_loop(0, nch, chunk, 0)
    plsc.subcore_barrier()

    @pl.when(s < 15)
    def _():
        pltpu.sync_copy(acc.at[pl.ds(s * ROWS_PT, ROWS_PT)],
                        out_hbm.at[c, pl.ds(s * ROWS_PT, ROWS_PT)])

    @pl.when(s == 15)
    def _():
        pltpu.sync_copy(acc.at[pl.ds(15 * ROWS_PT, N - 15 * ROWS_PT)],
                        out_hbm.at[c, pl.ds(15 * ROWS_PT, N - 15 * ROWS_PT)])


def _run_agg(p_flat, src, dst, ew, edge_split):
    ch = AGG_CH_ES if edge_split else AGG_CH_FS
    f = functools.partial(
        pl.kernel,
        mesh=_mesh,
        out_type=jax.ShapeDtypeStruct((2, N, 128), jnp.float32),
        scratch_types=[
            pltpu.VMEM((ch,), jnp.int32),
            pltpu.VMEM((ch,), jnp.int32),
            pltpu.VMEM((ch,), jnp.float32),
            pltpu.VMEM((W_WIN,), jnp.int32),
            pltpu.VMEM((W_WIN, 128), jnp.float32),
            pltpu.VMEM((W_WIN, 128), jnp.float32),
            pltpu.VMEM_SHARED((NPAD, 128), jnp.float32),
            pltpu.SemaphoreType.DMA,
            pltpu.SemaphoreType.DMA,
        ],
    )(functools.partial(_agg_body, edge_split))
    return f(p_flat, src, dst, ew)


_RB = 1000


def _dinv_body(degp_ref, out_ref):
    out_ref[...] = lax.rsqrt(degp_ref[0] + degp_ref[1] + 1.0)


def _run_dinv(degp):
    out = pl.pallas_call(
        _dinv_body,
        in_specs=[pl.BlockSpec((2, NPAD // 128, 128), lambda: (0, 0, 0))],
        out_specs=pl.BlockSpec((NPAD // 128, 128), lambda: (0, 0)),
        out_shape=jax.ShapeDtypeStruct((NPAD // 128, 128), jnp.float32),
    )(degp.reshape(2, NPAD // 128, 128))
    return out.reshape(NPAD, 1)


def _stage1_body(x_ref, wt_ref, bt_ref, w1_ref, dinv_ref, p1_ref):
    h0 = jnp.maximum(
        jnp.dot(x_ref[...], wt_ref[...], preferred_element_type=jnp.float32)
        + bt_ref[...], 0.0)
    g = jnp.dot(h0, w1_ref[...], preferred_element_type=jnp.float32)
    p = dinv_ref[...] * g
    p1_ref[0] = p[:, :128]
    p1_ref[1] = p[:, 128:]


def _stage2_body(a1_ref, p1_ref, b1_ref, w2t_ref, w2b_ref, dinv_ref, p2_ref):
    dinv = dinv_ref[...]
    h_lo = jnp.maximum(dinv * (a1_ref[0] + p1_ref[0]) + b1_ref[0, :][None, :], 0.0)
    h_hi = jnp.maximum(dinv * (a1_ref[1] + p1_ref[1]) + b1_ref[1, :][None, :], 0.0)
    g = (jnp.dot(h_lo, w2t_ref[...], preferred_element_type=jnp.float32)
         + jnp.dot(h_hi, w2b_ref[...], preferred_element_type=jnp.float32))
    p2_ref[...] = dinv * g


def _stage3_body(a2_ref, p2_ref, b2_ref, wc_ref, bc_ref, dinv_ref, out_ref):
    dinv = dinv_ref[...]
    agg0 = a2_ref[0] + a2_ref[1]
    h2 = jnp.maximum(dinv * (agg0 + p2_ref[...]) + b2_ref[...], 0.0)
    out_ref[...] = (
        jnp.dot(h2, wc_ref[...], preferred_element_type=jnp.float32)
        + bc_ref[...])


def _full(shape):
    return pl.BlockSpec(shape, lambda i: tuple(0 for _ in shape))


def _run_stage1(x, W_t, b_t, W1, dinv):
    grid = (N // _RB,)
    return pl.pallas_call(
        _stage1_body,
        grid=grid,
        in_specs=[
            pl.BlockSpec((_RB, 128), lambda i: (i, 0)),
            _full((128, 128)),
            _full((1, 128)),
            _full((128, 256)),
            pl.BlockSpec((_RB, 1), lambda i: (i, 0)),
        ],
        out_specs=pl.BlockSpec((2, _RB, 128), lambda i: (0, i, 0)),
        out_shape=jax.ShapeDtypeStruct((2, N, 128), jnp.float32),
    )(x, W_t, b_t.reshape(1, 128), W1, dinv)


def _run_stage2(a1, p1, b1, W2, dinv):
    grid = (N // _RB,)
    return pl.pallas_call(
        _stage2_body,
        grid=grid,
        in_specs=[
            pl.BlockSpec((2, _RB, 128), lambda i: (0, i, 0)),
            pl.BlockSpec((2, _RB, 128), lambda i: (0, i, 0)),
            _full((2, 128)),
            _full((128, 128)),
            _full((128, 128)),
            pl.BlockSpec((_RB, 1), lambda i: (i, 0)),
        ],
        out_specs=pl.BlockSpec((_RB, 128), lambda i: (i, 0)),
        out_shape=jax.ShapeDtypeStruct((N, 128), jnp.float32),
    )(a1, p1, b1.reshape(2, 128), W2[:128], W2[128:], dinv)


def _run_stage3(a2, p2, b2, Wc, bc, dinv):
    grid = (N // _RB,)
    return pl.pallas_call(
        _stage3_body,
        grid=grid,
        in_specs=[
            pl.BlockSpec((2, _RB, 128), lambda i: (0, i, 0)),
            pl.BlockSpec((_RB, 128), lambda i: (i, 0)),
            _full((1, 128)),
            _full((128, 8)),
            _full((1, 8)),
            pl.BlockSpec((_RB, 1), lambda i: (i, 0)),
        ],
        out_specs=pl.BlockSpec((_RB, 8), lambda i: (i, 0)),
        out_shape=jax.ShapeDtypeStruct((N, 8), jnp.float32),
    )(a2, p2, b2.reshape(1, 128), Wc, bc.reshape(1, 8), dinv)


@jax.jit
def kernel(x, edge_index, edge_attr, W_t, b_t, W1, b1, W2, b2, Wc, bc):
    ea = edge_attr.reshape(E)
    src = edge_index[0]
    dst = edge_index[1]
    ew, degp = _run_deg(ea, dst)
    dinv = _run_dinv(degp)
    p1 = _run_stage1(x, W_t, b_t, W1, dinv)
    a1 = _run_agg(p1.reshape(2 * N, 128), src, dst, ew, edge_split=False)
    p2 = _run_stage2(a1, p1, b1, W2, dinv)
    a2 = _run_agg(p2, src, dst, ew, edge_split=True)
    return _run_stage3(a2, p2, b2, Wc, bc, dinv)
